# baseline (speedup 1.0000x reference)
import jax
import jax.numpy as jnp
from jax.experimental import pallas as pl
from jax.experimental.pallas import tpu as pltpu

_B, _C, _T = 16, 1024, 4096
_R = _B * _C
_ROWS_BLK = 824


def _shift_body(cache_ref, x_ref, out_ref):
    blk = cache_ref[...]
    out_ref[...] = jnp.concatenate([blk[:, 1:], x_ref[...]], axis=1)


def kernel(cache, x):
    cache2 = cache.reshape(_R, _T)
    x2 = x.reshape(_R, 1)
    out = pl.pallas_call(
        _shift_body,
        grid=(pl.cdiv(_R, _ROWS_BLK),),
        in_specs=[
            pl.BlockSpec((_ROWS_BLK, _T), lambda i: (i, 0),
                         ),
            pl.BlockSpec((_ROWS_BLK, 1), lambda i: (i, 0),
                         ),
        ],
        out_specs=pl.BlockSpec((_ROWS_BLK, _T), lambda i: (i, 0),
                               ),
        out_shape=jax.ShapeDtypeStruct((_R, _T), cache.dtype),
    )(cache2, x2)
    return out.reshape(_B, _C, _T)
